# Initial kernel scaffold; baseline (speedup 1.0000x reference)
import jax
import jax.numpy as jnp
from jax.experimental import pallas as pl


def kernel(h, edge_index, coord, edge_attr, W_e1, b_e1, W_e2, b_e2, W_n1, b_n1, W_n2, b_n2, W_c1, b_c1, W_c2, b_c2):
    raise NotImplementedError("write your pallas kernel here")


# speedup vs baseline: 4.7125x; 4.7125x over previous
import functools

import jax
import jax.numpy as jnp
from jax import lax
from jax.experimental import pallas as pl
from jax.experimental.pallas import tpu as pltpu
from jax.experimental.pallas import tpu_sc as plsc

N = 10000
E = 320000
D = 128
DE = 16
H = 128

EW = 16
NC = 2
NS = 16
NW = NC * NS
CH = 128
NCHUNK = E // CH
ZR = 80
NZC = N // ZR

BN = 1000
BE = 1280


def _tables_body(h_ref, w1s_ref, w1t_ref, tabs_ref, tabt_ref):
    hb = h_ref[...]
    tabs_ref[...] = jnp.dot(hb, w1s_ref[...], preferred_element_type=jnp.float32)
    tabt_ref[...] = jnp.dot(hb, w1t_ref[...], preferred_element_type=jnp.float32)


def _build_tables(h, w1s, w1t):
    return pl.pallas_call(
        _tables_body,
        grid=(N // BN,),
        in_specs=[
            pl.BlockSpec((BN, D), lambda i: (i, 0)),
            pl.BlockSpec((D, H), lambda i: (0, 0)),
            pl.BlockSpec((D, H), lambda i: (0, 0)),
        ],
        out_specs=[
            pl.BlockSpec((BN, H), lambda i: (i, 0)),
            pl.BlockSpec((BN, H), lambda i: (i, 0)),
        ],
        out_shape=[
            jax.ShapeDtypeStruct((N, H), jnp.float32),
            jax.ShapeDtypeStruct((N, H), jnp.float32),
        ],
    )(h, w1s, w1t)


def _gather_body(tabs_hbm, tabt_hbm, cx_hbm, cy_hbm, cz_hbm, row_hbm, col_hbm,
                 pre_hbm, cd_hbm,
                 ir0, ir1, ic0, ic1, bs0, bs1, bt0, bt1, cb0, cb1,
                 cxv, cyv, czv,
                 si0, si1, sg0, sg1, so0, so1):
    wid = lax.axis_index("s") * NC + lax.axis_index("c")
    nmine = (NCHUNK - 1 - wid) // NW + 1

    irs = (ir0, ir1)
    ics = (ic0, ic1)
    bss = (bs0, bs1)
    bts = (bt0, bt1)
    cbs = (cb0, cb1)
    sis = (si0, si1)
    sgs = (sg0, sg1)
    sos = (so0, so1)

    pltpu.sync_copy(cx_hbm, cxv)
    pltpu.sync_copy(cy_hbm, cyv)
    pltpu.sync_copy(cz_hbm, czv)

    zv = jnp.zeros((16,), jnp.float32)

    def zrow(r, _):
        cb0[r, pl.ds(0, EW)] = zv
        cb1[r, pl.ds(0, EW)] = zv
        return 0

    lax.fori_loop(0, CH, zrow, 0)

    lanes = lax.iota(jnp.int32, 16)

    def issue_idx(t, sl):
        base = (wid + t * NW) * CH
        pltpu.async_copy(row_hbm.at[pl.ds(base, CH)], irs[sl], sis[sl])
        pltpu.async_copy(col_hbm.at[pl.ds(base, CH)], ics[sl], sis[sl])

    def wait_outs(sl):
        pltpu.make_async_copy(bss[sl], pre_hbm.at[pl.ds(0, CH)], sos[sl]).wait()
        pltpu.make_async_copy(cbs[sl], cd_hbm.at[pl.ds(0, CH)], sos[sl]).wait()

    def fire_gathers(t, sl):
        @pl.when(t >= 2)
        def _():
            wait_outs(sl)

        pltpu.make_async_copy(row_hbm.at[pl.ds(0, CH)], irs[sl], sis[sl]).wait()
        pltpu.make_async_copy(col_hbm.at[pl.ds(0, CH)], ics[sl], sis[sl]).wait()
        pltpu.async_copy(tabs_hbm.at[irs[sl]], bss[sl], sgs[sl])
        pltpu.async_copy(tabt_hbm.at[ics[sl]], bts[sl], sgs[sl])

    def finish(t, sl):
        base = (wid + t * NW) * CH
        pltpu.make_async_copy(tabs_hbm.at[irs[sl]], bss[sl], sgs[sl]).wait()
        pltpu.make_async_copy(tabt_hbm.at[ics[sl]], bts[sl], sgs[sl]).wait()
        bs = bss[sl]
        bt = bts[sl]
        cb = cbs[sl]
        ir = irs[sl]
        ic = ics[sl]

        def addrow(r, _):
            for j in range(H // 16):
                v = bt[r, pl.ds(j * 16, 16)]
                plsc.addupdate(bs.at[r, pl.ds(j * 16, 16)], v)
            return 0

        lax.fori_loop(0, CH, addrow, 0)

        def coordgrp(k, _):
            iv = ir[pl.ds(k * 16, 16)]
            jv = ic[pl.ds(k * 16, 16)]
            dx = plsc.load_gather(cxv, [iv]) - plsc.load_gather(cxv, [jv])
            dy = plsc.load_gather(cyv, [iv]) - plsc.load_gather(cyv, [jv])
            dz = plsc.load_gather(czv, [iv]) - plsc.load_gather(czv, [jv])
            rad = dx * dx + dy * dy + dz * dz
            rows = k * 16 + lanes
            plsc.store_scatter(cb, [rows, jnp.zeros((16,), jnp.int32)], dx)
            plsc.store_scatter(cb, [rows, jnp.ones((16,), jnp.int32)], dy)
            plsc.store_scatter(cb, [rows, jnp.full((16,), 2, jnp.int32)], dz)
            plsc.store_scatter(cb, [rows, jnp.full((16,), 3, jnp.int32)], rad)
            return 0

        lax.fori_loop(0, CH // 16, coordgrp, 0)
        pltpu.async_copy(bs, pre_hbm.at[pl.ds(base, CH)], sos[sl])
        pltpu.async_copy(cb, cd_hbm.at[pl.ds(base, CH)], sos[sl])

    issue_idx(0, 0)
    fire_gathers(0, 0)

    @pl.when(nmine >= 2)
    def _():
        issue_idx(1, 1)

    def pair(p, _):
        t0 = 2 * p
        t1 = t0 + 1

        @pl.when(t1 < nmine)
        def _():
            fire_gathers(t1, 1)

        finish(t0, 0)

        @pl.when(t0 + 2 < nmine)
        def _():
            issue_idx(t0 + 2, 0)

        @pl.when(t1 < nmine)
        def _():
            finish(t1, 1)

        @pl.when(t1 + 2 < nmine)
        def _():
            issue_idx(t1 + 2, 1)

        @pl.when(t0 + 2 < nmine)
        def _():
            fire_gathers(t0 + 2, 0)

        return 0

    lax.fori_loop(0, (nmine + 1) // 2, pair, 0)
    wait_outs(0)
    wait_outs(1)


def _gather(tabs, tabt, cx, cy, cz, row, col):
    mesh = plsc.VectorSubcoreMesh(core_axis_name="c", subcore_axis_name="s")
    f = functools.partial(
        pl.kernel,
        out_type=[
            jax.ShapeDtypeStruct((E, H), jnp.float32),
            jax.ShapeDtypeStruct((E, EW), jnp.float32),
        ],
        mesh=mesh,
        scratch_types=[
            pltpu.VMEM((CH,), jnp.int32),
            pltpu.VMEM((CH,), jnp.int32),
            pltpu.VMEM((CH,), jnp.int32),
            pltpu.VMEM((CH,), jnp.int32),
            pltpu.VMEM((CH, H), jnp.float32),
            pltpu.VMEM((CH, H), jnp.float32),
            pltpu.VMEM((CH, H), jnp.float32),
            pltpu.VMEM((CH, H), jnp.float32),
            pltpu.VMEM((CH, EW), jnp.float32),
            pltpu.VMEM((CH, EW), jnp.float32),
            pltpu.VMEM((N,), jnp.float32),
            pltpu.VMEM((N,), jnp.float32),
            pltpu.VMEM((N,), jnp.float32),
            pltpu.SemaphoreType.DMA,
            pltpu.SemaphoreType.DMA,
            pltpu.SemaphoreType.DMA,
            pltpu.SemaphoreType.DMA,
            pltpu.SemaphoreType.DMA,
            pltpu.SemaphoreType.DMA,
        ],
        compiler_params=pltpu.CompilerParams(needs_layout_passes=False),
    )(_gather_body)
    return f(tabs, tabt, cx, cy, cz, row, col)


def _edge_body(pre_ref, cd_ref, attr_ref, w1a_ref, w1r_ref, be1_ref, we2_ref,
               be2_ref, wc1_ref, bc1_ref, wc2_ref, bc2_ref,
               mij_ref, ext_ref):
    cd = cd_ref[...]
    radial = cd[:, 3:4]
    m1 = pre_ref[...] + radial * w1r_ref[...] + be1_ref[...]
    m1 = m1 + jnp.dot(attr_ref[...], w1a_ref[...],
                      preferred_element_type=jnp.float32)
    m1 = jnp.maximum(m1, 0.0)
    mij = jnp.dot(m1, we2_ref[...], preferred_element_type=jnp.float32)
    mij = jnp.maximum(mij + be2_ref[...], 0.0)
    mij_ref[...] = mij
    c1 = jnp.maximum(jnp.dot(mij, wc1_ref[...],
                             preferred_element_type=jnp.float32)
                     + bc1_ref[...], 0.0)
    cw = jnp.sum(c1 * wc2_ref[...], axis=1, keepdims=True) + bc2_ref[...]
    lane = lax.broadcasted_iota(jnp.int32, cd.shape, 1)
    ext_ref[...] = jnp.where(lane == 3, 1.0, cd * cw)


def _edge_mlp(pre, cd, edge_attr, w1a, w1r, b_e1, W_e2, b_e2, W_c1, b_c1,
              wc2r, b_c2):
    full = lambda shape: pl.BlockSpec(shape, lambda i: tuple(0 for _ in shape))
    return pl.pallas_call(
        _edge_body,
        grid=(E // BE,),
        in_specs=[
            pl.BlockSpec((BE, H), lambda i: (i, 0)),
            pl.BlockSpec((BE, EW), lambda i: (i, 0)),
            pl.BlockSpec((BE, DE), lambda i: (i, 0)),
            full((DE, H)), full((1, H)), full((1, H)), full((H, H)),
            full((1, H)), full((H, H)), full((1, H)), full((1, H)),
            full((1, 1)),
        ],
        out_specs=[
            pl.BlockSpec((BE, H), lambda i: (i, 0)),
            pl.BlockSpec((BE, EW), lambda i: (i, 0)),
        ],
        out_shape=[
            jax.ShapeDtypeStruct((E, H), jnp.float32),
            jax.ShapeDtypeStruct((E, EW), jnp.float32),
        ],
    )(pre, cd, edge_attr, w1a, w1r, b_e1, W_e2, b_e2, W_c1, b_c1, wc2r, b_c2)


def _scatter_pipeline(dat_hbm, row_hbm, acc_sh, idxs, bufs, sls, sss, nmine, s):
    def load(t, sl):
        @pl.when(t >= 2)
        def _():
            pltpu.make_async_copy(bufs[sl], acc_sh.at[idxs[sl]], sss[sl]).wait()

        base = (s + t * NS) * CH
        pltpu.async_copy(row_hbm.at[pl.ds(base, CH)], idxs[sl], sls[sl])
        pltpu.async_copy(dat_hbm.at[pl.ds(base, CH)], bufs[sl], sls[sl])

    def scat(t, sl):
        pltpu.make_async_copy(row_hbm.at[pl.ds(0, CH)], idxs[sl], sls[sl]).wait()
        pltpu.make_async_copy(dat_hbm.at[pl.ds(0, CH)], bufs[sl], sls[sl]).wait()
        pltpu.async_copy(bufs[sl], acc_sh.at[idxs[sl]], sss[sl], add=True)

    load(0, 0)

    @pl.when(nmine >= 2)
    def _():
        load(1, 1)

    def pair(p, _):
        t0 = 2 * p
        t1 = t0 + 1
        scat(t0, 0)

        @pl.when(t0 + 2 < nmine)
        def _():
            load(t0 + 2, 0)

        @pl.when(t1 < nmine)
        def _():
            scat(t1, 1)

        @pl.when(t1 + 2 < nmine)
        def _():
            load(t1 + 2, 1)

        return 0

    lax.fori_loop(0, (nmine + 1) // 2, pair, 0)
    pltpu.make_async_copy(bufs[0], acc_sh.at[idxs[0]], sss[0]).wait()
    pltpu.make_async_copy(bufs[1], acc_sh.at[idxs[1]], sss[1]).wait()


def _scatter_agg_body(mij_hbm, row_hbm, agg_hbm,
                      i0, i1, b0, b1, zbuf, agg_sh, sl0, sl1, ss0, ss1):
    s = lax.axis_index("s")
    nmine = (NCHUNK - 1 - s) // NS + 1

    zv = jnp.zeros((16,), jnp.float32)

    def zrow(r, _):
        for j in range(H // 16):
            zbuf[r, pl.ds(j * 16, 16)] = zv
        return 0

    lax.fori_loop(0, ZR, zrow, 0)

    for k in range(NZC // NS + 1):
        cidx = s + k * NS

        @pl.when(cidx < NZC)
        def _():
            pltpu.sync_copy(zbuf, agg_sh.at[pl.ds(cidx * ZR, ZR)])

    plsc.subcore_barrier()
    _scatter_pipeline(mij_hbm, row_hbm, agg_sh, (i0, i1), (b0, b1),
                      (sl0, sl1), (ss0, ss1), nmine, s)
    plsc.subcore_barrier()
    for k in range(NZC // NS + 1):
        cidx = s + k * NS

        @pl.when(cidx < NZC)
        def _():
            pltpu.sync_copy(agg_sh.at[pl.ds(cidx * ZR, ZR)],
                            agg_hbm.at[pl.ds(cidx * ZR, ZR)])


def _scatter_agg(mij, row):
    mesh = plsc.VectorSubcoreMesh(core_axis_name="c", subcore_axis_name="s",
                                  num_cores=1)
    f = functools.partial(
        pl.kernel,
        out_type=jax.ShapeDtypeStruct((N, H), jnp.float32),
        mesh=mesh,
        scratch_types=[
            pltpu.VMEM((CH,), jnp.int32),
            pltpu.VMEM((CH,), jnp.int32),
            pltpu.VMEM((CH, H), jnp.float32),
            pltpu.VMEM((CH, H), jnp.float32),
            pltpu.VMEM((ZR, H), jnp.float32),
            pltpu.VMEM_SHARED((N, H), jnp.float32),
            pltpu.SemaphoreType.DMA,
            pltpu.SemaphoreType.DMA,
            pltpu.SemaphoreType.DMA,
            pltpu.SemaphoreType.DMA,
        ],
    )(_scatter_agg_body)
    return f(mij, row)


def _scatter_ext_body(ext_hbm, row_hbm, exto_hbm,
                      i0, i1, b0, b1, zeb, ext_sh, sl0, sl1, ss0, ss1):
    s = lax.axis_index("s")
    nmine = (NCHUNK - 1 - s) // NS + 1

    zv = jnp.zeros((16,), jnp.float32)

    def zerow(r, _):
        zeb[r, pl.ds(0, EW)] = zv
        return 0

    lax.fori_loop(0, ZR, zerow, 0)
    for k in range(NZC // NS + 1):
        cidx = s + k * NS

        @pl.when(cidx < NZC)
        def _():
            pltpu.sync_copy(zeb, ext_sh.at[pl.ds(cidx * ZR, ZR)])

    plsc.subcore_barrier()
    _scatter_pipeline(ext_hbm, row_hbm, ext_sh, (i0, i1), (b0, b1),
                      (sl0, sl1), (ss0, ss1), nmine, s)
    plsc.subcore_barrier()
    for k in range(NZC // NS + 1):
        cidx = s + k * NS

        @pl.when(cidx < NZC)
        def _():
            pltpu.sync_copy(ext_sh.at[pl.ds(cidx * ZR, ZR)],
                            exto_hbm.at[pl.ds(cidx * ZR, ZR)])


def _scatter_ext(ext, row):
    mesh = plsc.VectorSubcoreMesh(core_axis_name="c", subcore_axis_name="s",
                                  num_cores=1)
    f = functools.partial(
        pl.kernel,
        out_type=jax.ShapeDtypeStruct((N, EW), jnp.float32),
        mesh=mesh,
        scratch_types=[
            pltpu.VMEM((CH,), jnp.int32),
            pltpu.VMEM((CH,), jnp.int32),
            pltpu.VMEM((CH, EW), jnp.float32),
            pltpu.VMEM((CH, EW), jnp.float32),
            pltpu.VMEM((ZR, EW), jnp.float32),
            pltpu.VMEM_SHARED((N, EW), jnp.float32),
            pltpu.SemaphoreType.DMA,
            pltpu.SemaphoreType.DMA,
            pltpu.SemaphoreType.DMA,
            pltpu.SemaphoreType.DMA,
        ],
    )(_scatter_ext_body)
    return f(ext, row)


def _node_body(h_ref, aggp_ref, extp_ref, coord_ref, wn1h_ref, wn1a_ref,
               bn1_ref, wn2_ref, bn2_ref, hout_ref, cout_ref):
    agg = aggp_ref[...]
    x1 = jnp.dot(h_ref[...], wn1h_ref[...], preferred_element_type=jnp.float32)
    x1 = x1 + jnp.dot(agg, wn1a_ref[...], preferred_element_type=jnp.float32)
    x1 = jnp.maximum(x1 + bn1_ref[...], 0.0)
    hout_ref[...] = jnp.dot(x1, wn2_ref[...],
                            preferred_element_type=jnp.float32) + bn2_ref[...]
    ext = extp_ref[...]
    cnt = jnp.maximum(ext[:, 3:4], 1.0)
    cout_ref[...] = coord_ref[...] + ext[:, :3] / cnt


def _node_mlp(h, aggp, extp, coord, wn1h, wn1a, b_n1, W_n2, b_n2):
    full = lambda shape: pl.BlockSpec(shape, lambda i: tuple(0 for _ in shape))
    return pl.pallas_call(
        _node_body,
        grid=(N // BN,),
        in_specs=[
            pl.BlockSpec((BN, D), lambda i: (i, 0)),
            pl.BlockSpec((BN, H), lambda i: (i, 0)),
            pl.BlockSpec((BN, EW), lambda i: (i, 0)),
            pl.BlockSpec((BN, 3), lambda i: (i, 0)),
            full((D, H)), full((H, H)), full((1, H)), full((H, H)),
            full((1, H)),
        ],
        out_specs=[
            pl.BlockSpec((BN, H), lambda i: (i, 0)),
            pl.BlockSpec((BN, 3), lambda i: (i, 0)),
        ],
        out_shape=[
            jax.ShapeDtypeStruct((N, H), jnp.float32),
            jax.ShapeDtypeStruct((N, 3), jnp.float32),
        ],
    )(h, aggp, extp, coord, wn1h, wn1a, b_n1, W_n2, b_n2)


def kernel(h, edge_index, coord, edge_attr, W_e1, b_e1, W_e2, b_e2,
           W_n1, b_n1, W_n2, b_n2, W_c1, b_c1, W_c2, b_c2):
    row = edge_index[0]
    col = edge_index[1]
    w1s = W_e1[:D]
    w1t = W_e1[D:2 * D]
    w1r = W_e1[2 * D:2 * D + 1]
    w1a = W_e1[2 * D + 1:]
    cx = coord[:, 0]
    cy = coord[:, 1]
    cz = coord[:, 2]

    tabs, tabt = _build_tables(h, w1s, w1t)
    pre, cd = _gather(tabs, tabt, cx, cy, cz, row, col)
    mij, ext = _edge_mlp(
        pre, cd, edge_attr, w1a, w1r, b_e1.reshape(1, H), W_e2,
        b_e2.reshape(1, H), W_c1, b_c1.reshape(1, H),
        W_c2.reshape(1, H), b_c2.reshape(1, 1))
    aggp = _scatter_agg(mij, row)
    extp = _scatter_ext(ext, row)
    h_out, coord_out = _node_mlp(
        h, aggp, extp, coord, W_n1[:D], W_n1[D:], b_n1.reshape(1, H),
        W_n2, b_n2.reshape(1, H))
    return (h_out, coord_out, mij)
